# baseline (speedup 1.0000x reference)
import jax
import jax.numpy as jnp
from jax.experimental import pallas as pl
from jax.experimental.pallas import tpu as pltpu

_BSZ = 128
_REP = 8
_NCHUNK = _BSZ // _REP


def _body(t_ref, o_ref, buf, sems, stage_sem):
    nd = t_ref.shape[0]
    for r in range(_REP):
        pltpu.make_async_copy(
            t_ref, buf.at[pl.ds(r * nd, nd)], stage_sem
        ).start()
    for r in range(_REP):
        pltpu.make_async_copy(
            t_ref, buf.at[pl.ds(r * nd, nd)], stage_sem
        ).wait()
    chunk = _REP * nd
    for i in range(_NCHUNK):
        pltpu.make_async_copy(
            buf, o_ref.at[pl.ds(i * chunk, chunk)], sems.at[i]
        ).start()
    for i in range(_NCHUNK):
        pltpu.make_async_copy(
            buf, o_ref.at[pl.ds(i * chunk, chunk)], sems.at[i]
        ).wait()


def kernel(batch_size, table):
    n, d = table.shape
    nd = n * d
    flat = table.reshape(nd)
    out = pl.pallas_call(
        _body,
        in_specs=[pl.BlockSpec(memory_space=pltpu.VMEM)],
        out_specs=pl.BlockSpec(memory_space=pltpu.HBM),
        out_shape=jax.ShapeDtypeStruct((_BSZ * nd,), table.dtype),
        scratch_shapes=[
            pltpu.VMEM((_REP * nd,), table.dtype),
            pltpu.SemaphoreType.DMA((_NCHUNK,)),
            pltpu.SemaphoreType.DMA,
        ],
    )(flat)
    return out.reshape(_BSZ, n, d)


# speedup vs baseline: 1.3061x; 1.3061x over previous
import jax
import jax.numpy as jnp
from jax.experimental import pallas as pl
from jax.experimental.pallas import tpu as pltpu

_BSZ = 128
_NBUF = 4
_NSEM = 16


def _body(t_ref, o_ref, b0, b1, b2, b3, sems, stage_sem):
    bufs = [b0, b1, b2, b3]
    for b in bufs:
        pltpu.make_async_copy(t_ref, b, stage_sem).start()
    for b in bufs:
        pltpu.make_async_copy(t_ref, b, stage_sem).wait()
    copies = [
        pltpu.make_async_copy(bufs[i % _NBUF], o_ref.at[i], sems.at[i % _NSEM])
        for i in range(_BSZ)
    ]
    for c in copies:
        c.start()
    for c in copies:
        c.wait()


def kernel(batch_size, table):
    n, d = table.shape
    return pl.pallas_call(
        _body,
        in_specs=[pl.BlockSpec(memory_space=pltpu.VMEM)],
        out_specs=pl.BlockSpec(memory_space=pltpu.HBM),
        out_shape=jax.ShapeDtypeStruct((_BSZ, n, d), table.dtype),
        scratch_shapes=[
            pltpu.VMEM((n, d), table.dtype),
            pltpu.VMEM((n, d), table.dtype),
            pltpu.VMEM((n, d), table.dtype),
            pltpu.VMEM((n, d), table.dtype),
            pltpu.SemaphoreType.DMA((_NSEM,)),
            pltpu.SemaphoreType.DMA,
        ],
    )(table)
